# baseline (speedup 1.0000x reference)
import jax
import jax.numpy as jnp
from jax import lax
from jax.experimental import pallas as pl
from jax.experimental.pallas import tpu as pltpu
from jax.experimental.pallas import tpu_sc as plsc

N_GROUPS = 5000
N_USERS = 5000
N_ITEMS = 5000
HID = 128
N_NODES = 10000
LAYERS = 3
E = 320000

NC = 2
NT = 16
NPAD = 10240
RPT = NPAD // NT
WBC = 16
NWB = RPT // WBC
PAD_ROWS = NPAD - N_NODES

CHUNK = 128
BLK = 8
NBLK = 20
EPT_CHUNKS = BLK * NBLK
EPT = EPT_CHUNKS * CHUNK
EPAD = EPT * NT

GCH = 64
NSLOT = 4
GPB = BLK * CHUNK // GCH
LOOKAHEAD = 3


def _rsqrt_newton(d):
    i = lax.bitcast_convert_type(d, jnp.int32)
    i = jnp.int32(0x5F3759DF) - (i >> 1)
    y = lax.bitcast_convert_type(i, jnp.float32)
    for _ in range(3):
        y = y * (1.5 - 0.5 * d * y * y)
    return jnp.where(d > 0.5, y, 0.0)


def _sc_body(x_hbm, srcb_hbm, dstb_hbm, z_hbm,
             y_hbm, xs_hbm,
             acc_sh, odeg_sh, ideg_sh,
             sidx, didx, rows_buf, wb_buf, ones_buf,
             no_v, ni_v, w_v, dbuf, zvec,
             sem_g0, sem_g1, sem_g2, sem_g3,
             sem_s0, sem_s1, sem_s2, sem_s3, sem_si, sem_di):
    gsems = (sem_g0, sem_g1, sem_g2, sem_g3)
    ssems = (sem_s0, sem_s1, sem_s2, sem_s3)
    c = lax.axis_index("c")
    s = lax.axis_index("s")
    base = s * RPT
    gbase = c * NPAD + base
    tid = c * NT + s

    def _ones(j, carry):
        ones_buf[pl.ds(j * 16, 16)] = jnp.ones((16,), jnp.float32)
        return carry
    lax.fori_loop(0, BLK * CHUNK // 16, _ones, 0)
    for j in range(RPT // 16):
        zvec[pl.ds(j * 16, 16)] = jnp.zeros((16,), jnp.float32)

    def _sidx_copy(b, slot):
        return pltpu.make_async_copy(
            srcb_hbm.at[tid].at[b], sidx.at[slot], sem_si)

    def _didx_copy(b, slot):
        return pltpu.make_async_copy(
            dstb_hbm.at[tid].at[b], didx.at[slot], sem_di)

    pltpu.sync_copy(zvec, odeg_sh.at[pl.ds(c * NPAD + base, RPT)])
    pltpu.sync_copy(zvec, ideg_sh.at[pl.ds(base, RPT)])
    plsc.subcore_barrier()

    _sidx_copy(0, 0).start()
    _didx_copy(0, 0).start()

    def _degblk(b, carry):
        slot = b % 2
        _sidx_copy(b, slot).wait()
        _didx_copy(b, slot).wait()

        @pl.when(b + 1 < NBLK)
        def _():
            _sidx_copy(b + 1, 1 - slot).start()
            _didx_copy(b + 1, 1 - slot).start()
        pltpu.sync_copy(ones_buf, odeg_sh.at[sidx.at[slot]], add=True)
        pltpu.sync_copy(ones_buf, ideg_sh.at[didx.at[slot]], add=True)
        return carry
    lax.fori_loop(0, NBLK, _degblk, 0)
    plsc.subcore_barrier()

    pltpu.sync_copy(odeg_sh.at[pl.ds(c * NPAD + base, RPT)], dbuf)

    def _no(j, carry):
        no_v[pl.ds(j * 16, 16)] = _rsqrt_newton(dbuf[pl.ds(j * 16, 16)])
        return carry
    lax.fori_loop(0, RPT // 16, _no, 0)
    pltpu.sync_copy(ideg_sh.at[pl.ds(base, RPT)], dbuf)

    def _ni(j, carry):
        ni_v[pl.ds(j * 16, 16)] = _rsqrt_newton(dbuf[pl.ds(j * 16, 16)])
        return carry
    lax.fori_loop(0, RPT // 16, _ni, 0)

    def _w(j, carry):
        w_v[pl.ds(j * 16, 16)] = no_v[pl.ds(j * 16, 16)] * ni_v[pl.ds(j * 16, 16)]
        return carry
    lax.fori_loop(0, RPT // 16, _w, 0)

    def _scale_rows(scale_v, off):
        mvec = scale_v[pl.ds(off, 16)]
        for rr in range(WBC):
            vm = jnp.broadcast_to(mvec[rr], (16,))
            for col in range(HID // 16):
                wb_buf[rr, pl.ds(col * 16, 16)] = (
                    wb_buf[rr, pl.ds(col * 16, 16)] * vm)

    def _init_scale(wc, carry):
        pltpu.sync_copy(x_hbm.at[pl.ds(gbase + wc * WBC, WBC)], wb_buf)
        _scale_rows(no_v, wc * WBC)
        pltpu.sync_copy(wb_buf, xs_hbm.at[pl.ds(gbase + wc * WBC, WBC)])
        return carry
    lax.fori_loop(0, NWB, _init_scale, 0)
    plsc.subcore_barrier()

    for layer in range(LAYERS):
        pltpu.sync_copy(z_hbm.at[pl.ds(base, RPT)],
                        acc_sh.at[pl.ds(base, RPT)])
        plsc.subcore_barrier()

        _sidx_copy(0, 0).start()
        _didx_copy(0, 0).start()

        def _blk(b, carry):
            slotI = b % 2
            _sidx_copy(b, slotI).wait()
            _didx_copy(b, slotI).wait()

            @pl.when(b + 1 < NBLK)
            def _():
                _sidx_copy(b + 1, 1 - slotI).start()
                _didx_copy(b + 1, 1 - slotI).start()

            si = sidx.at[slotI]
            di = didx.at[slotI]

            def _g(j, slot):
                return pltpu.make_async_copy(
                    xs_hbm.at[si.at[pl.ds(j * GCH, GCH)]],
                    rows_buf.at[slot], gsems[slot])

            def _s(j, slot):
                return pltpu.make_async_copy(
                    rows_buf.at[slot],
                    acc_sh.at[di.at[pl.ds(j * GCH, GCH)]], ssems[slot])

            def _s_start(j, slot):
                pltpu.async_copy(
                    rows_buf.at[slot],
                    acc_sh.at[di.at[pl.ds(j * GCH, GCH)]], ssems[slot],
                    add=True)

            for j in range(LOOKAHEAD):
                _g(j, j % NSLOT).start()
            for j in range(GPB):
                slot = j % NSLOT
                if j + LOOKAHEAD < GPB:
                    ns = (j + LOOKAHEAD) % NSLOT
                    if j + LOOKAHEAD >= NSLOT:
                        _s(j + LOOKAHEAD - NSLOT, ns).wait()
                    _g(j + LOOKAHEAD, ns).start()
                _g(j, slot).wait()
                _s_start(j, slot)
            for k in range(GPB - NSLOT, GPB):
                _s(k, k % NSLOT).wait()
            return carry
        lax.fori_loop(0, NBLK, _blk, 0)
        plsc.subcore_barrier()

        last = layer == LAYERS - 1
        scale_v = ni_v if last else w_v
        target = y_hbm if last else xs_hbm

        def _wb(wc, carry):
            pltpu.sync_copy(acc_sh.at[pl.ds(base + wc * WBC, WBC)], wb_buf)
            _scale_rows(scale_v, wc * WBC)
            pltpu.sync_copy(wb_buf, target.at[pl.ds(gbase + wc * WBC, WBC)])
            return carry
        lax.fori_loop(0, NWB, _wb, 0)
        plsc.subcore_barrier()


def _build_call():
    mesh = plsc.VectorSubcoreMesh(core_axis_name="c", subcore_axis_name="s",
                                  num_cores=NC, num_subcores=NT)
    return pl.kernel(
        _sc_body,
        out_type=(
            jax.ShapeDtypeStruct((NC * NPAD, HID), jnp.float32),
            jax.ShapeDtypeStruct((NC * NPAD, HID), jnp.float32),
        ),
        mesh=mesh,
        scratch_types=[
            pltpu.VMEM_SHARED((NPAD, HID), jnp.float32),
            pltpu.VMEM_SHARED((NC * NPAD,), jnp.float32),
            pltpu.VMEM_SHARED((NPAD,), jnp.float32),
            pltpu.VMEM((2, BLK * CHUNK), jnp.int32),
            pltpu.VMEM((2, BLK * CHUNK), jnp.int32),
            pltpu.VMEM((NSLOT, GCH, HID), jnp.float32),
            pltpu.VMEM((WBC, HID), jnp.float32),
            pltpu.VMEM((BLK * CHUNK,), jnp.float32),
            pltpu.VMEM((RPT,), jnp.float32),
            pltpu.VMEM((RPT,), jnp.float32),
            pltpu.VMEM((RPT,), jnp.float32),
            pltpu.VMEM((RPT,), jnp.float32),
            pltpu.VMEM((RPT,), jnp.float32),
            pltpu.SemaphoreType.DMA,
            pltpu.SemaphoreType.DMA,
            pltpu.SemaphoreType.DMA,
            pltpu.SemaphoreType.DMA,
            pltpu.SemaphoreType.DMA,
            pltpu.SemaphoreType.DMA,
            pltpu.SemaphoreType.DMA,
            pltpu.SemaphoreType.DMA,
            pltpu.SemaphoreType.DMA,
            pltpu.SemaphoreType.DMA,
        ],
        compiler_params=pltpu.CompilerParams(use_tc_tiling_on_sc=False),
    )


def kernel(group_emb, user_emb, item_emb, ui_edge_index, gi_edge_index):
    x0 = jnp.concatenate([user_emb, item_emb], axis=0)
    x1 = jnp.concatenate([group_emb, item_emb], axis=0)
    X = jnp.zeros((NC * NPAD, HID), jnp.float32)
    X = X.at[:N_NODES].set(x0).at[NPAD:NPAD + N_NODES].set(x1)

    pad = N_NODES + (jnp.arange(EPAD - E, dtype=jnp.int32) % PAD_ROWS)

    def prep(ei, g):
        src = jnp.concatenate([ei[0].astype(jnp.int32), pad]) + g * NPAD
        dst = jnp.concatenate([ei[1].astype(jnp.int32), pad])
        return src, dst

    s0, d0 = prep(ui_edge_index, 0)
    s1, d1 = prep(gi_edge_index, 1)
    srcb = jnp.stack([s0, s1]).reshape(NC * NT, NBLK, BLK * CHUNK)
    dstb = jnp.stack([d0, d1]).reshape(NC * NT, NBLK, BLK * CHUNK)
    zeros = jnp.zeros((NPAD, HID), jnp.float32)

    y, _ = _build_call()(X, srcb, dstb, zeros)

    h_user = y[:N_USERS]
    h_item = jnp.concatenate(
        [y[N_USERS:N_NODES], jnp.zeros((1, HID), jnp.float32)], axis=0)
    h_group = y[NPAD:NPAD + N_GROUPS]
    return (h_group, h_user, h_item)


# speedup vs baseline: 1.0408x; 1.0408x over previous
import jax
import jax.numpy as jnp
from jax import lax
from jax.experimental import pallas as pl
from jax.experimental.pallas import tpu as pltpu
from jax.experimental.pallas import tpu_sc as plsc

N_GROUPS = 5000
N_USERS = 5000
N_ITEMS = 5000
HID = 128
N_NODES = 10000
LAYERS = 3
E = 320000

NC = 2
NT = 16
NPAD = 10240
RPT = NPAD // NT
WBC = 16
NWB = RPT // WBC
PAD_ROWS = NPAD - N_NODES

CHUNK = 128
BLK = 8
NBLK = 20
EPT_CHUNKS = BLK * NBLK
EPT = EPT_CHUNKS * CHUNK
EPAD = EPT * NT

GCH = 64
NSLOT = 4
GPB = BLK * CHUNK // GCH
LOOKAHEAD = 3


def _rsqrt_newton(d):
    i = lax.bitcast_convert_type(d, jnp.int32)
    i = jnp.int32(0x5F3759DF) - (i >> 1)
    y = lax.bitcast_convert_type(i, jnp.float32)
    for _ in range(3):
        y = y * (1.5 - 0.5 * d * y * y)
    return jnp.where(d > 0.5, y, 0.0)


def _sc_body(x_hbm, srcb_hbm, dstb_hbm, z_hbm,
             y_hbm, xs_hbm,
             acc_sh, odeg_sh, ideg_sh,
             sidx, didx, rows_buf, wb_buf, ones_buf,
             no_v, ni_v, w_v, dbuf, zvec,
             sem_g0, sem_g1, sem_g2, sem_g3,
             sem_s0, sem_s1, sem_s2, sem_s3, sem_si, sem_di,
             sem_wi0, sem_wi1, sem_wo0, sem_wo1):
    gsems = (sem_g0, sem_g1, sem_g2, sem_g3)
    ssems = (sem_s0, sem_s1, sem_s2, sem_s3)
    c = lax.axis_index("c")
    s = lax.axis_index("s")
    base = s * RPT
    gbase = c * NPAD + base
    tid = c * NT + s

    def _ones(j, carry):
        ones_buf[pl.ds(j * 16, 16)] = jnp.ones((16,), jnp.float32)
        return carry
    lax.fori_loop(0, BLK * CHUNK // 16, _ones, 0)
    for j in range(RPT // 16):
        zvec[pl.ds(j * 16, 16)] = jnp.zeros((16,), jnp.float32)

    def _sidx_copy(b, slot):
        return pltpu.make_async_copy(
            srcb_hbm.at[tid].at[b], sidx.at[slot], sem_si)

    def _didx_copy(b, slot):
        return pltpu.make_async_copy(
            dstb_hbm.at[tid].at[b], didx.at[slot], sem_di)

    pltpu.sync_copy(zvec, odeg_sh.at[pl.ds(c * NPAD + base, RPT)])
    pltpu.sync_copy(zvec, ideg_sh.at[pl.ds(base, RPT)])
    plsc.subcore_barrier()

    _sidx_copy(0, 0).start()
    _didx_copy(0, 0).start()

    def _degblk(b, carry):
        slot = b % 2
        _sidx_copy(b, slot).wait()
        _didx_copy(b, slot).wait()

        @pl.when(b + 1 < NBLK)
        def _():
            _sidx_copy(b + 1, 1 - slot).start()
            _didx_copy(b + 1, 1 - slot).start()
        pltpu.async_copy(ones_buf, odeg_sh.at[sidx.at[slot]], sem_wo0,
                         add=True)
        pltpu.async_copy(ones_buf, ideg_sh.at[didx.at[slot]], sem_wo1,
                         add=True)
        pltpu.make_async_copy(ones_buf, odeg_sh.at[sidx.at[slot]],
                              sem_wo0).wait()
        pltpu.make_async_copy(ones_buf, ideg_sh.at[didx.at[slot]],
                              sem_wo1).wait()
        return carry
    lax.fori_loop(0, NBLK, _degblk, 0)
    plsc.subcore_barrier()

    pltpu.sync_copy(odeg_sh.at[pl.ds(c * NPAD + base, RPT)], dbuf)

    def _no(j, carry):
        no_v[pl.ds(j * 16, 16)] = _rsqrt_newton(dbuf[pl.ds(j * 16, 16)])
        return carry
    lax.fori_loop(0, RPT // 16, _no, 0)
    pltpu.sync_copy(ideg_sh.at[pl.ds(base, RPT)], dbuf)

    def _ni(j, carry):
        ni_v[pl.ds(j * 16, 16)] = _rsqrt_newton(dbuf[pl.ds(j * 16, 16)])
        return carry
    lax.fori_loop(0, RPT // 16, _ni, 0)

    def _w(j, carry):
        w_v[pl.ds(j * 16, 16)] = no_v[pl.ds(j * 16, 16)] * ni_v[pl.ds(j * 16, 16)]
        return carry
    lax.fori_loop(0, RPT // 16, _w, 0)

    def _scale_rows(slot, scale_v, off):
        mvec = scale_v[pl.ds(off, 16)]
        for rr in range(WBC):
            vm = jnp.broadcast_to(mvec[rr], (16,))
            for col in range(HID // 16):
                wb_buf[slot, rr, pl.ds(col * 16, 16)] = (
                    wb_buf[slot, rr, pl.ds(col * 16, 16)] * vm)

    def _scale_phase(src_ref, src_base, dst_ref, dst_base, scale_v):
        wsems = (sem_wi0, sem_wi1)
        osems = (sem_wo0, sem_wo1)

        def _in(wc, slot):
            return pltpu.make_async_copy(
                src_ref.at[pl.ds(src_base + wc * WBC, WBC)],
                wb_buf.at[slot], wsems[slot])

        def _out(wc, slot):
            return pltpu.make_async_copy(
                wb_buf.at[slot],
                dst_ref.at[pl.ds(dst_base + wc * WBC, WBC)], osems[slot])

        _in(0, 0).start()

        def _pair(i, carry):
            wc = i * 2
            _in(wc, 0).wait()

            @pl.when(i >= 1)
            def _():
                _out(wc - 1, 1).wait()
            _in(wc + 1, 1).start()
            _scale_rows(0, scale_v, wc * WBC)
            _out(wc, 0).start()

            _in(wc + 1, 1).wait()

            @pl.when(i + 1 < NWB // 2)
            def _():
                _out(wc, 0).wait()
                _in(wc + 2, 0).start()
            _scale_rows(1, scale_v, (wc + 1) * WBC)
            _out(wc + 1, 1).start()
            return carry
        lax.fori_loop(0, NWB // 2, _pair, 0)
        _out(NWB - 2, 0).wait()
        _out(NWB - 1, 1).wait()

    _scale_phase(x_hbm, gbase, xs_hbm, gbase, no_v)
    plsc.subcore_barrier()

    for layer in range(LAYERS):
        pltpu.sync_copy(z_hbm.at[pl.ds(base, RPT)],
                        acc_sh.at[pl.ds(base, RPT)])
        plsc.subcore_barrier()

        _sidx_copy(0, 0).start()
        _didx_copy(0, 0).start()

        def _blk(b, carry):
            slotI = b % 2
            _sidx_copy(b, slotI).wait()
            _didx_copy(b, slotI).wait()

            @pl.when(b + 1 < NBLK)
            def _():
                _sidx_copy(b + 1, 1 - slotI).start()
                _didx_copy(b + 1, 1 - slotI).start()

            si = sidx.at[slotI]
            di = didx.at[slotI]

            def _g(j, slot):
                return pltpu.make_async_copy(
                    xs_hbm.at[si.at[pl.ds(j * GCH, GCH)]],
                    rows_buf.at[slot], gsems[slot])

            def _s(j, slot):
                return pltpu.make_async_copy(
                    rows_buf.at[slot],
                    acc_sh.at[di.at[pl.ds(j * GCH, GCH)]], ssems[slot])

            def _s_start(j, slot):
                pltpu.async_copy(
                    rows_buf.at[slot],
                    acc_sh.at[di.at[pl.ds(j * GCH, GCH)]], ssems[slot],
                    add=True)

            for j in range(LOOKAHEAD):
                _g(j, j % NSLOT).start()
            for j in range(GPB):
                slot = j % NSLOT
                if j + LOOKAHEAD < GPB:
                    ns = (j + LOOKAHEAD) % NSLOT
                    if j + LOOKAHEAD >= NSLOT:
                        _s(j + LOOKAHEAD - NSLOT, ns).wait()
                    _g(j + LOOKAHEAD, ns).start()
                _g(j, slot).wait()
                _s_start(j, slot)
            for k in range(GPB - NSLOT, GPB):
                _s(k, k % NSLOT).wait()
            return carry
        lax.fori_loop(0, NBLK, _blk, 0)
        plsc.subcore_barrier()

        last = layer == LAYERS - 1
        scale_v = ni_v if last else w_v
        target = y_hbm if last else xs_hbm

        _scale_phase(acc_sh, base, target, gbase, scale_v)
        plsc.subcore_barrier()


def _build_call():
    mesh = plsc.VectorSubcoreMesh(core_axis_name="c", subcore_axis_name="s",
                                  num_cores=NC, num_subcores=NT)
    return pl.kernel(
        _sc_body,
        out_type=(
            jax.ShapeDtypeStruct((NC * NPAD, HID), jnp.float32),
            jax.ShapeDtypeStruct((NC * NPAD, HID), jnp.float32),
        ),
        mesh=mesh,
        scratch_types=[
            pltpu.VMEM_SHARED((NPAD, HID), jnp.float32),
            pltpu.VMEM_SHARED((NC * NPAD,), jnp.float32),
            pltpu.VMEM_SHARED((NPAD,), jnp.float32),
            pltpu.VMEM((2, BLK * CHUNK), jnp.int32),
            pltpu.VMEM((2, BLK * CHUNK), jnp.int32),
            pltpu.VMEM((NSLOT, GCH, HID), jnp.float32),
            pltpu.VMEM((2, WBC, HID), jnp.float32),
            pltpu.VMEM((BLK * CHUNK,), jnp.float32),
            pltpu.VMEM((RPT,), jnp.float32),
            pltpu.VMEM((RPT,), jnp.float32),
            pltpu.VMEM((RPT,), jnp.float32),
            pltpu.VMEM((RPT,), jnp.float32),
            pltpu.VMEM((RPT,), jnp.float32),
            pltpu.SemaphoreType.DMA,
            pltpu.SemaphoreType.DMA,
            pltpu.SemaphoreType.DMA,
            pltpu.SemaphoreType.DMA,
            pltpu.SemaphoreType.DMA,
            pltpu.SemaphoreType.DMA,
            pltpu.SemaphoreType.DMA,
            pltpu.SemaphoreType.DMA,
            pltpu.SemaphoreType.DMA,
            pltpu.SemaphoreType.DMA,
            pltpu.SemaphoreType.DMA,
            pltpu.SemaphoreType.DMA,
            pltpu.SemaphoreType.DMA,
            pltpu.SemaphoreType.DMA,
        ],
        compiler_params=pltpu.CompilerParams(use_tc_tiling_on_sc=False),
    )


def kernel(group_emb, user_emb, item_emb, ui_edge_index, gi_edge_index):
    x0 = jnp.concatenate([user_emb, item_emb], axis=0)
    x1 = jnp.concatenate([group_emb, item_emb], axis=0)
    X = jnp.zeros((NC * NPAD, HID), jnp.float32)
    X = X.at[:N_NODES].set(x0).at[NPAD:NPAD + N_NODES].set(x1)

    pad = N_NODES + (jnp.arange(EPAD - E, dtype=jnp.int32) % PAD_ROWS)

    def prep(ei, g):
        src = jnp.concatenate([ei[0].astype(jnp.int32), pad]) + g * NPAD
        dst = jnp.concatenate([ei[1].astype(jnp.int32), pad])
        return src, dst

    s0, d0 = prep(ui_edge_index, 0)
    s1, d1 = prep(gi_edge_index, 1)
    srcb = jnp.stack([s0, s1]).reshape(NC * NT, NBLK, BLK * CHUNK)
    dstb = jnp.stack([d0, d1]).reshape(NC * NT, NBLK, BLK * CHUNK)
    zeros = jnp.zeros((NPAD, HID), jnp.float32)

    y, _ = _build_call()(X, srcb, dstb, zeros)

    h_user = y[:N_USERS]
    h_item = jnp.concatenate(
        [y[N_USERS:N_NODES], jnp.zeros((1, HID), jnp.float32)], axis=0)
    h_group = y[NPAD:NPAD + N_GROUPS]
    return (h_group, h_user, h_item)
